# baseline (speedup 1.0000x reference)
import functools

import jax
import jax.numpy as jnp
from jax import lax
from jax.experimental import pallas as pl
from jax.experimental.pallas import tpu as pltpu
from jax.experimental.pallas import tpu_sc as plsc

B = 16384
D = 128
NC = 2
NS = 16
NW = NC * NS
NSPLIT = 2
BS = B // NSPLIT
CMAX = 256

_mesh = plsc.VectorSubcoreMesh(core_axis_name="c", subcore_axis_name="s")


def _make_gather(bs):
    bpw = bs // NW
    c = min(bpw, CMAX)
    nchunk = bpw // c

    @functools.partial(
        pl.kernel,
        mesh=_mesh,
        out_type=(
            jax.ShapeDtypeStruct((bs, D), jnp.float32),
            jax.ShapeDtypeStruct((bs, D), jnp.float32),
            jax.ShapeDtypeStruct((bs,), jnp.float32),
        ),
        compiler_params=pltpu.CompilerParams(needs_layout_passes=False),
        scratch_types=[
            pltpu.VMEM((bpw,), jnp.int32),
            pltpu.VMEM((bpw,), jnp.int32),
            pltpu.VMEM((c, D), jnp.float32),
            pltpu.VMEM((c, D), jnp.float32),
            pltpu.VMEM((bpw,), jnp.float32),
            pltpu.SemaphoreType.DMA,
            pltpu.SemaphoreType.DMA,
            pltpu.SemaphoreType.DMA,
        ],
    )
    def gather_sc(sid_hbm, qid_hbm, se_hbm, df_hbm, disc_hbm,
                  seg_hbm, dfg_hbm, discg_hbm,
                  sid_v, qid_v, se_v, df_v, disc_v,
                  sem_se, sem_df, sem_disc):
        wid = lax.axis_index("s") * NC + lax.axis_index("c")
        base = wid * bpw

        pltpu.sync_copy(sid_hbm.at[pl.ds(base, bpw)], sid_v)
        pltpu.sync_copy(qid_hbm.at[pl.ds(base, bpw)], qid_v)
        cp_disc = pltpu.async_copy(disc_hbm.at[qid_v], disc_v, sem_disc)

        for i in range(nchunk):
            cp_se = pltpu.async_copy(
                se_hbm.at[sid_v.at[pl.ds(i * c, c)]], se_v, sem_se)
            cp_df = pltpu.async_copy(
                df_hbm.at[qid_v.at[pl.ds(i * c, c)]], df_v, sem_df)
            cp_se.wait()
            pltpu.sync_copy(se_v, seg_hbm.at[pl.ds(base + i * c, c)])
            cp_df.wait()
            pltpu.sync_copy(df_v, dfg_hbm.at[pl.ds(base + i * c, c)])

        cp_disc.wait()
        pltpu.sync_copy(disc_v, discg_hbm.at[pl.ds(base, bpw)])

    return gather_sc


def _tc_body(seg_ref, dfg_ref, q_ref, disc_ref, out_ref):
    prof = jax.nn.sigmoid(seg_ref[...])
    diff = jax.nn.sigmoid(dfg_ref[...])
    s = jnp.sum((prof - diff) * q_ref[...], axis=1, keepdims=True)
    out_ref[...] = jax.nn.sigmoid(jax.nn.sigmoid(disc_ref[...]) * s)


def _make_tc(bs, bb=2048):
    return pl.pallas_call(
        _tc_body,
        grid=(bs // bb,),
        in_specs=[
            pl.BlockSpec((bb, D), lambda i: (i, 0)),
            pl.BlockSpec((bb, D), lambda i: (i, 0)),
            pl.BlockSpec((bb, D), lambda i: (i, 0)),
            pl.BlockSpec((bb, 1), lambda i: (i, 0)),
        ],
        out_specs=pl.BlockSpec((bb, 1), lambda i: (i, 0)),
        out_shape=jax.ShapeDtypeStruct((bs, 1), jnp.float32),
    )


_gather_slice = _make_gather(BS)
_tc_slice = _make_tc(BS)


def kernel(student_id, question, q_matrix_line, student_emb, difficulty, discrimination):
    sid = student_id.astype(jnp.int32)
    qid = question.astype(jnp.int32)
    disc1 = discrimination.reshape(-1)
    outs = []
    for s in range(NSPLIT):
        sl = slice(s * BS, (s + 1) * BS)
        seg, dfg, discg = _gather_slice(sid[sl], qid[sl], student_emb,
                                        difficulty, disc1)
        outs.append(_tc_slice(seg, dfg, q_matrix_line[sl], discg.reshape(BS, 1)))
    return jnp.concatenate(outs, axis=0).reshape(-1)


# speedup vs baseline: 1.1146x; 1.1146x over previous
import functools

import jax
import jax.numpy as jnp
from jax import lax
from jax.experimental import pallas as pl
from jax.experimental.pallas import tpu as pltpu
from jax.experimental.pallas import tpu_sc as plsc

B = 16384
D = 128
NC = 2
NS = 16
NW = NC * NS
NSPLIT = 1
BS = B // NSPLIT
CMAX = 256

_mesh = plsc.VectorSubcoreMesh(core_axis_name="c", subcore_axis_name="s")


def _make_gather(bs):
    bpw = bs // NW
    c = min(bpw, CMAX)
    nchunk = bpw // c

    @functools.partial(
        pl.kernel,
        mesh=_mesh,
        out_type=(
            jax.ShapeDtypeStruct((bs, D), jnp.float32),
            jax.ShapeDtypeStruct((bs, D), jnp.float32),
            jax.ShapeDtypeStruct((bs,), jnp.float32),
        ),
        compiler_params=pltpu.CompilerParams(
            needs_layout_passes=False, skip_device_barrier=True),
        scratch_types=[
            pltpu.VMEM((bpw,), jnp.int32),
            pltpu.VMEM((bpw,), jnp.int32),
            pltpu.VMEM((c, D), jnp.float32),
            pltpu.VMEM((c, D), jnp.float32),
            pltpu.VMEM((bpw,), jnp.float32),
            pltpu.SemaphoreType.DMA,
            pltpu.SemaphoreType.DMA,
            pltpu.SemaphoreType.DMA,
        ],
    )
    def gather_sc(sid_hbm, qid_hbm, se_hbm, df_hbm, disc_hbm,
                  seg_hbm, dfg_hbm, discg_hbm,
                  sid_v, qid_v, se_v, df_v, disc_v,
                  sem_se, sem_df, sem_disc):
        wid = lax.axis_index("s") * NC + lax.axis_index("c")
        base = wid * bpw

        pltpu.sync_copy(sid_hbm.at[pl.ds(base, bpw)], sid_v)
        pltpu.sync_copy(qid_hbm.at[pl.ds(base, bpw)], qid_v)
        cp_disc = pltpu.async_copy(disc_hbm.at[qid_v], disc_v, sem_disc)

        for i in range(nchunk):
            cp_se = pltpu.async_copy(
                se_hbm.at[sid_v.at[pl.ds(i * c, c)]], se_v, sem_se)
            cp_df = pltpu.async_copy(
                df_hbm.at[qid_v.at[pl.ds(i * c, c)]], df_v, sem_df)
            cp_se.wait()
            pltpu.sync_copy(se_v, seg_hbm.at[pl.ds(base + i * c, c)])
            cp_df.wait()
            pltpu.sync_copy(df_v, dfg_hbm.at[pl.ds(base + i * c, c)])

        cp_disc.wait()
        pltpu.sync_copy(disc_v, discg_hbm.at[pl.ds(base, bpw)])

    return gather_sc


def _tc_body(seg_ref, dfg_ref, q_ref, disc_ref, out_ref):
    prof = jax.nn.sigmoid(seg_ref[...])
    diff = jax.nn.sigmoid(dfg_ref[...])
    s = jnp.sum((prof - diff) * q_ref[...], axis=1, keepdims=True)
    out_ref[...] = jax.nn.sigmoid(jax.nn.sigmoid(disc_ref[...]) * s)


def _make_tc(bs, bb=4096):
    return pl.pallas_call(
        _tc_body,
        grid=(bs // bb,),
        in_specs=[
            pl.BlockSpec((bb, D), lambda i: (i, 0)),
            pl.BlockSpec((bb, D), lambda i: (i, 0)),
            pl.BlockSpec((bb, D), lambda i: (i, 0)),
            pl.BlockSpec((bb, 1), lambda i: (i, 0)),
        ],
        out_specs=pl.BlockSpec((bb, 1), lambda i: (i, 0)),
        out_shape=jax.ShapeDtypeStruct((bs, 1), jnp.float32),
    )


_gather_slice = _make_gather(BS)
_tc_slice = _make_tc(BS)


def kernel(student_id, question, q_matrix_line, student_emb, difficulty, discrimination):
    sid = student_id.astype(jnp.int32)
    qid = question.astype(jnp.int32)
    disc1 = discrimination.reshape(-1)
    outs = []
    for s in range(NSPLIT):
        sl = slice(s * BS, (s + 1) * BS)
        seg, dfg, discg = _gather_slice(sid[sl], qid[sl], student_emb,
                                        difficulty, disc1)
        outs.append(_tc_slice(seg, dfg, q_matrix_line[sl], discg.reshape(BS, 1)))
    return jnp.concatenate(outs, axis=0).reshape(-1)


# speedup vs baseline: 1.4210x; 1.2749x over previous
import functools

import jax
import jax.numpy as jnp
from jax import lax
from jax.experimental import pallas as pl
from jax.experimental.pallas import tpu as pltpu
from jax.experimental.pallas import tpu_sc as plsc

B = 16384
D = 128
NC = 2
NS = 16
NW = NC * NS
NSPLIT = 1
BS = B // NSPLIT
CMAX = 256

_mesh = plsc.VectorSubcoreMesh(core_axis_name="c", subcore_axis_name="s")


def _make_gather(bs):
    bpw = bs // NW
    c = min(bpw, CMAX)
    nchunk = bpw // c

    @functools.partial(
        pl.kernel,
        mesh=_mesh,
        out_type=(
            jax.ShapeDtypeStruct((bs, D), jnp.float32),
            jax.ShapeDtypeStruct((bs, D), jnp.float32),
            jax.ShapeDtypeStruct((bs,), jnp.float32),
        ),
        compiler_params=pltpu.CompilerParams(
            needs_layout_passes=False, skip_device_barrier=True),
        scratch_types=[
            pltpu.VMEM((bpw,), jnp.int32),
            pltpu.VMEM((bpw,), jnp.int32),
            pltpu.VMEM((c, D), jnp.float32),
            pltpu.VMEM((c, D), jnp.float32),
            pltpu.VMEM((bpw,), jnp.float32),
            pltpu.SemaphoreType.DMA,
            pltpu.SemaphoreType.DMA,
            pltpu.SemaphoreType.DMA,
        ],
    )
    def gather_sc(sid_hbm, qid_hbm, se_hbm, df_hbm, disc_hbm,
                  seg_hbm, dfg_hbm, discg_hbm,
                  sid_v, qid_v, se_v, df_v, disc_v,
                  sem_se, sem_df, sem_disc):
        wid = lax.axis_index("s") * NC + lax.axis_index("c")
        base = wid * bpw

        pltpu.sync_copy(sid_hbm.at[pl.ds(base, bpw)], sid_v)
        pltpu.sync_copy(qid_hbm.at[pl.ds(base, bpw)], qid_v)
        cp_disc = pltpu.async_copy(disc_hbm.at[qid_v], disc_v, sem_disc)

        for i in range(nchunk):
            cp_se = pltpu.async_copy(
                se_hbm.at[sid_v.at[pl.ds(i * c, c)]], se_v, sem_se)
            cp_df = pltpu.async_copy(
                df_hbm.at[qid_v.at[pl.ds(i * c, c)]], df_v, sem_df)
            cp_se.wait()
            pltpu.sync_copy(se_v, seg_hbm.at[pl.ds(base + i * c, c)])
            cp_df.wait()
            pltpu.sync_copy(df_v, dfg_hbm.at[pl.ds(base + i * c, c)])

        cp_disc.wait()
        pltpu.sync_copy(disc_v, discg_hbm.at[pl.ds(base, bpw)])

    return gather_sc


def _tc_body(seg_ref, dfg_ref, q_ref, disc_ref, out_ref):
    prof = jax.nn.sigmoid(seg_ref[...])
    diff = jax.nn.sigmoid(dfg_ref[...])
    s = jnp.sum((prof - diff) * q_ref[...], axis=1)
    out_ref[...] = jax.nn.sigmoid(jax.nn.sigmoid(disc_ref[...]) * s)


def _make_tc(bs, bb=4096):
    return pl.pallas_call(
        _tc_body,
        grid=(bs // bb,),
        in_specs=[
            pl.BlockSpec((bb, D), lambda i: (i, 0)),
            pl.BlockSpec((bb, D), lambda i: (i, 0)),
            pl.BlockSpec((bb, D), lambda i: (i, 0)),
            pl.BlockSpec((bb,), lambda i: (i,)),
        ],
        out_specs=pl.BlockSpec((bb,), lambda i: (i,)),
        out_shape=jax.ShapeDtypeStruct((bs,), jnp.float32),
    )


_gather_slice = _make_gather(BS)
_tc_slice = _make_tc(BS)


def kernel(student_id, question, q_matrix_line, student_emb, difficulty, discrimination):
    sid = student_id.astype(jnp.int32)
    qid = question.astype(jnp.int32)
    disc1 = discrimination.reshape(-1)
    outs = []
    for s in range(NSPLIT):
        sl = slice(s * BS, (s + 1) * BS)
        seg, dfg, discg = _gather_slice(sid[sl], qid[sl], student_emb,
                                        difficulty, disc1)
        outs.append(_tc_slice(seg, dfg, q_matrix_line[sl], discg))
    if NSPLIT == 1:
        return outs[0]
    return jnp.concatenate(outs, axis=0)
